# baseline (speedup 1.0000x reference)
import functools

import jax
import jax.numpy as jnp
from jax import lax
from jax.experimental import pallas as pl
from jax.experimental.pallas import tpu as pltpu
import jax.experimental.pallas.tpu_sc as plsc

F32 = jnp.float32
HID = 128
H2 = HID // 2
NW = 32
CHUNK = 128
BLK_E = 2048
BLK_N = 1024


def _cdiv(a, b):
  return (a + b - 1) // b


def _rup(a, b):
  return _cdiv(a, b) * b


def _silu(h):
  return h * jax.nn.sigmoid(h)


def _ln(h, g, b):
  mu = jnp.mean(h, axis=-1, keepdims=True)
  d = h - mu
  var = jnp.mean(d * d, axis=-1, keepdims=True)
  return d * lax.rsqrt(var + 1e-5) * g + b


def _dot(a, b):
  return jnp.dot(a.astype(jnp.bfloat16), b.astype(jnp.bfloat16),
                 preferred_element_type=F32)


def _rows(body, row_ins, w_ins, out_dims, blk):
  nrows = row_ins[0].shape[0]
  grid = (nrows // blk,)
  in_specs = (
      [pl.BlockSpec((blk, a.shape[1]), lambda i: (i, 0)) for a in row_ins]
      + [pl.BlockSpec(w.shape, lambda i, nd=w.ndim: (0,) * nd) for w in w_ins]
  )
  out_specs = [pl.BlockSpec((blk, d), lambda i: (i, 0)) for d in out_dims]
  out_shape = [jax.ShapeDtypeStruct((nrows, d), F32) for d in out_dims]
  outs = pl.pallas_call(
      body, grid=grid, in_specs=in_specs, out_specs=out_specs,
      out_shape=out_shape)(*row_ins, *w_ins)
  return outs


def _enc_body(x_ref, w0, b0, w1, b1, w2, b2, g, b, out_ref):
  h = _silu(_dot(x_ref[...], w0[...]) + b0[...])
  h = _silu(_dot(h, w1[...]) + b1[...])
  h = _dot(h, w2[...]) + b2[...]
  out_ref[...] = _ln(h, g[...], b[...])


def _make_cb_body(skip):
  def body(*refs):
    xh_ref, a0_ref, a1_ref = refs[:3]
    rest = refs[3:]
    if skip:
      x0_ref, rest = rest[0], rest[1:]
    w0x, w0a, b0, w1, b1, w2, b2, g, b, w0s, w0d = rest[:11]
    xh_out, gs_out, gd_out = rest[11:]
    xh = xh_ref[...]
    agg = a0_ref[...] + a1_ref[...]
    aggf = 0.5 * (agg[:, :H2] + agg[:, H2:])
    h = _silu(_dot(xh, w0x[...]) + _dot(aggf, w0a[...]) + b0[...])
    h = _silu(_dot(h, w1[...]) + b1[...])
    h = _dot(h, w2[...]) + b2[...]
    xn = _ln(h, g[...], b[...])
    out = xh + xn
    if skip:
      out = out + x0_ref[...]
    xh_out[...] = out
    gs_out[...] = _dot(xn, w0s[...])
    gd_out[...] = _dot(xn, w0d[...])
  return body


def _make_edge_body(skip):
  def body(*refs):
    eh_ref, gs_ref, gd_ref = refs[:3]
    rest = refs[3:]
    if skip:
      e0_ref, rest = rest[0], rest[1:]
    w0e, b0, w1, b1, w2, b2, g, b = rest[:8]
    out_ref = rest[8]
    eh = eh_ref[...]
    h = _silu(gs_ref[...] + gd_ref[...] + _dot(eh, w0e[...]) + b0[...])
    h = _silu(_dot(h, w1[...]) + b1[...])
    h = _dot(h, w2[...]) + b2[...]
    en = _ln(h, g[...], b[...])
    out = eh + en
    if skip:
      out = out + e0_ref[...]
    out_ref[...] = out
  return body


def _dec_body(eh_ref, w0, b0, w1, b1, w2, b2, out_ref):
  h = _silu(_dot(eh_ref[...], w0[...]) + b0[...])
  h = _silu(_dot(h, w1[...]) + b1[...])
  out_ref[...] = _dot(h, w2[...]) + b2[...]


def _sc_gather(gs, gd, srci, dsti, e_pad):
  ch = e_pad // (NW * CHUNK)
  per_w = ch * CHUNK
  mesh = plsc.VectorSubcoreMesh(core_axis_name="c", subcore_axis_name="s")

  @functools.partial(
      pl.kernel,
      out_type=[jax.ShapeDtypeStruct((e_pad, HID), F32)] * 2,
      mesh=mesh,
      scratch_types=[
          pltpu.VMEM((ch, CHUNK), jnp.int32),
          pltpu.VMEM((ch, CHUNK), jnp.int32),
          pltpu.VMEM((CHUNK, HID), F32),
          pltpu.VMEM((CHUNK, HID), F32),
          pltpu.SemaphoreType.DMA,
          pltpu.SemaphoreType.DMA,
      ])
  def k(gs_hbm, gd_hbm, si_hbm, di_hbm, os_hbm, od_hbm,
        si_v, di_v, as_v, ad_v, sem_a, sem_b):
    cid = lax.axis_index("c")
    sid = lax.axis_index("s")
    wid = sid * 2 + cid
    pltpu.sync_copy(si_hbm.at[wid], si_v)
    pltpu.sync_copy(di_hbm.at[wid], di_v)
    base = wid * per_w

    def step(j, carry):
      cs = pltpu.async_copy(gs_hbm.at[si_v.at[j]], as_v, sem_a)
      cd = pltpu.async_copy(gd_hbm.at[di_v.at[j]], ad_v, sem_b)
      cs.wait()
      cd.wait()
      ws = pltpu.make_async_copy(
          as_v, os_hbm.at[pl.ds(base + j * CHUNK, CHUNK)], sem_a)
      wd = pltpu.make_async_copy(
          ad_v, od_hbm.at[pl.ds(base + j * CHUNK, CHUNK)], sem_b)
      ws.start()
      wd.start()
      ws.wait()
      wd.wait()
      return carry

    lax.fori_loop(0, ch, step, 0)

  return k(gs, gd, srci, dsti)


def _sc_scatter(rows, dsti, zeros_n, n_pad, e_pad):
  ch = e_pad // (NW * CHUNK)
  per_w = ch * CHUNK
  mesh = plsc.VectorSubcoreMesh(core_axis_name="c", subcore_axis_name="s")

  @functools.partial(
      pl.kernel,
      out_type=jax.ShapeDtypeStruct((2, n_pad, HID), F32),
      mesh=mesh,
      scratch_types=[
          pltpu.VMEM((CHUNK,), jnp.int32),
          pltpu.VMEM((CHUNK, HID), F32),
          pltpu.VMEM_SHARED((n_pad, HID), F32),
          pltpu.SemaphoreType.DMA,
          pltpu.SemaphoreType.DMA,
      ])
  def k(r_hbm, di_hbm, z_hbm, out_hbm, di_v, fb_v, acc_sh, sem_a, sem_b):
    cid = lax.axis_index("c")
    sid = lax.axis_index("s")
    wid = sid * 2 + cid

    @pl.when(sid == 0)
    def _():
      pltpu.sync_copy(z_hbm, acc_sh)

    plsc.subcore_barrier()
    base = wid * per_w

    def step(j, carry):
      ra = pltpu.async_copy(r_hbm.at[pl.ds(base + j * CHUNK, CHUNK)], fb_v,
                            sem_a)
      ri = pltpu.async_copy(di_hbm.at[wid, j], di_v, sem_b)
      ra.wait()
      ri.wait()
      pltpu.sync_copy(fb_v, acc_sh.at[di_v], add=True)
      return carry

    lax.fori_loop(0, ch, step, 0)
    plsc.subcore_barrier()

    @pl.when(sid == 0)
    def _():
      pltpu.sync_copy(acc_sh, out_hbm.at[cid])

  return k(rows, dsti, zeros_n)


def kernel(x, edge_attr, edge_index, params):
  n = x.shape[0]
  e = edge_attr.shape[0]
  n_pad = _rup(n, BLK_N)
  e_pad = _rup(e, NW * CHUNK)
  pad_node = n_pad - 1

  src = edge_index[0].astype(jnp.int32)
  dst = edge_index[1].astype(jnp.int32)
  srci = jnp.full((e_pad,), pad_node, jnp.int32).at[:e].set(src)
  dsti = jnp.full((e_pad,), pad_node, jnp.int32).at[:e].set(dst)
  srci = srci.reshape(NW, -1, CHUNK)
  dsti = dsti.reshape(NW, -1, CHUNK)
  zeros_n = jnp.zeros((n_pad, HID), F32)

  x_p = jnp.zeros((n_pad, 8), F32).at[:n, :2].set(x)
  ea_p = jnp.zeros((e_pad, 8), F32).at[:e, :3].set(edge_attr)

  def prep(p, in_pad=None):
    w0 = p['w0']
    if in_pad is not None:
      w0 = jnp.zeros((in_pad, w0.shape[1]), F32).at[:w0.shape[0]].set(w0)
    out = [w0, p['b0'].reshape(1, -1), p['w1'], p['b1'].reshape(1, -1),
           p['w2'], p['b2'].reshape(1, -1)]
    if 'ln_g' in p:
      out += [p['ln_g'].reshape(1, -1), p['ln_b'].reshape(1, -1)]
    return out

  xh, = _rows(_enc_body, [x_p], prep(params['enc_cb'], 8), [HID], BLK_N)
  eh, = _rows(_enc_body, [ea_p], prep(params['enc_eb'], 8), [HID], BLK_E)
  x0, e0 = xh, eh

  mp = len(params['blocks'])
  for i, blk in enumerate(params['blocks']):
    skip = (i == mp - mp // 2)
    cbw = prep(blk['cb'])
    ebw = prep(blk['eb'])
    w0x, w0a = cbw[0][:HID], cbw[0][HID:]
    w0s, w0d, w0e = ebw[0][:HID], ebw[0][HID:2 * HID], ebw[0][2 * HID:]

    agg = _sc_scatter(eh, dsti, zeros_n, n_pad, e_pad)
    row_ins = [xh, agg[0], agg[1]] + ([x0] if skip else [])
    xh, gs, gd = _rows(
        _make_cb_body(skip), row_ins,
        [w0x, w0a] + cbw[1:] + [w0s, w0d], [HID, HID, HID], BLK_N)

    gth_s, gth_d = _sc_gather(gs, gd, srci, dsti, e_pad)

    row_ins = [eh, gth_s, gth_d] + ([e0] if skip else [])
    eh, = _rows(_make_edge_body(skip), row_ins, [w0e] + ebw[1:], [HID], BLK_E)

  dw = prep(params['dec'])
  odim = dw[4].shape[1]
  opad = _rup(odim, 8)
  dw[4] = jnp.zeros((HID, opad), F32).at[:, :odim].set(dw[4])
  dw[5] = jnp.zeros((1, opad), F32).at[:, :odim].set(dw[5])
  out_p, = _rows(_dec_body, [eh], dw, [opad], BLK_E)
  return out_p[:e, :odim]


# speedup vs baseline: 1.0376x; 1.0376x over previous
import functools

import jax
import jax.numpy as jnp
from jax import lax
from jax.experimental import pallas as pl
from jax.experimental.pallas import tpu as pltpu
import jax.experimental.pallas.tpu_sc as plsc

F32 = jnp.float32
HID = 128
H2 = HID // 2
NW = 32
CHUNK = 128
BLK_E = 2048
BLK_N = 1024


def _cdiv(a, b):
  return (a + b - 1) // b


def _rup(a, b):
  return _cdiv(a, b) * b


def _silu(h):
  return h * jax.nn.sigmoid(h)


def _ln(h, g, b):
  mu = jnp.mean(h, axis=-1, keepdims=True)
  d = h - mu
  var = jnp.mean(d * d, axis=-1, keepdims=True)
  return d * lax.rsqrt(var + 1e-5) * g + b


def _dot(a, b):
  return jnp.dot(a, b, preferred_element_type=F32)


def _rows(body, row_ins, w_ins, out_dims, blk):
  nrows = row_ins[0].shape[0]
  grid = (nrows // blk,)
  in_specs = (
      [pl.BlockSpec((blk, a.shape[1]), lambda i: (i, 0)) for a in row_ins]
      + [pl.BlockSpec(w.shape, lambda i, nd=w.ndim: (0,) * nd) for w in w_ins]
  )
  out_specs = [pl.BlockSpec((blk, d), lambda i: (i, 0)) for d in out_dims]
  out_shape = [jax.ShapeDtypeStruct((nrows, d), F32) for d in out_dims]
  outs = pl.pallas_call(
      body, grid=grid, in_specs=in_specs, out_specs=out_specs,
      out_shape=out_shape)(*row_ins, *w_ins)
  return outs


def _enc_body(x_ref, w0, b0, w1, b1, w2, b2, g, b, out_ref):
  h = _silu(_dot(x_ref[...], w0[...]) + b0[...])
  h = _silu(_dot(h, w1[...]) + b1[...])
  h = _dot(h, w2[...]) + b2[...]
  out_ref[...] = _ln(h, g[...], b[...])


def _make_cb_body(skip):
  def body(*refs):
    xh_ref, a0_ref, a1_ref = refs[:3]
    rest = refs[3:]
    if skip:
      x0_ref, rest = rest[0], rest[1:]
    w0x, w0a, b0, w1, b1, w2, b2, g, b, w0s, w0d = rest[:11]
    xh_out, gs_out, gd_out = rest[11:]
    xh = xh_ref[...]
    agg = a0_ref[...] + a1_ref[...]
    aggf = 0.5 * (agg[:, :H2] + agg[:, H2:])
    h = _silu(_dot(xh, w0x[...]) + _dot(aggf, w0a[...]) + b0[...])
    h = _silu(_dot(h, w1[...]) + b1[...])
    h = _dot(h, w2[...]) + b2[...]
    xn = _ln(h, g[...], b[...])
    out = xh + xn
    if skip:
      out = out + x0_ref[...]
    xh_out[...] = out
    gs_out[...] = _dot(xn, w0s[...])
    gd_out[...] = _dot(xn, w0d[...])
  return body


def _make_edge_body(skip, decode):
  def body(*refs):
    eh_ref, gs_ref, gd_ref = refs[:3]
    rest = refs[3:]
    if skip:
      e0_ref, rest = rest[0], rest[1:]
    w0e, b0, w1, b1, w2, b2, g, b = rest[:8]
    rest = rest[8:]
    eh = eh_ref[...]
    h = _silu(gs_ref[...] + gd_ref[...] + _dot(eh, w0e[...]) + b0[...])
    h = _silu(_dot(h, w1[...]) + b1[...])
    h = _dot(h, w2[...]) + b2[...]
    en = _ln(h, g[...], b[...])
    out = eh + en
    if skip:
      out = out + e0_ref[...]
    if decode:
      dw0, db0, dw1, db1, dw2, db2, out_ref = rest
      h = _silu(_dot(out, dw0[...]) + db0[...])
      h = _silu(_dot(h, dw1[...]) + db1[...])
      out_ref[...] = _dot(h, dw2[...]) + db2[...]
    else:
      rest[0][...] = out
  return body


def _dec_body(eh_ref, w0, b0, w1, b1, w2, b2, out_ref):
  h = _silu(_dot(eh_ref[...], w0[...]) + b0[...])
  h = _silu(_dot(h, w1[...]) + b1[...])
  out_ref[...] = _dot(h, w2[...]) + b2[...]


def _sc_gather(gs, gd, srci, dsti, e_pad):
  ch = e_pad // (NW * CHUNK)
  per_w = ch * CHUNK
  mesh = plsc.VectorSubcoreMesh(core_axis_name="c", subcore_axis_name="s")

  @functools.partial(
      pl.kernel,
      out_type=[jax.ShapeDtypeStruct((e_pad, HID), F32)] * 2,
      mesh=mesh,
      scratch_types=[
          pltpu.VMEM((ch, CHUNK), jnp.int32),
          pltpu.VMEM((ch, CHUNK), jnp.int32),
          pltpu.VMEM((CHUNK, HID), F32),
          pltpu.VMEM((CHUNK, HID), F32),
          pltpu.SemaphoreType.DMA,
          pltpu.SemaphoreType.DMA,
      ])
  def k(gs_hbm, gd_hbm, si_hbm, di_hbm, os_hbm, od_hbm,
        si_v, di_v, as_v, ad_v, sem_a, sem_b):
    cid = lax.axis_index("c")
    sid = lax.axis_index("s")
    wid = sid * 2 + cid
    pltpu.sync_copy(si_hbm.at[wid], si_v)
    pltpu.sync_copy(di_hbm.at[wid], di_v)
    base = wid * per_w

    def step(j, carry):
      cs = pltpu.async_copy(gs_hbm.at[si_v.at[j]], as_v, sem_a)
      cd = pltpu.async_copy(gd_hbm.at[di_v.at[j]], ad_v, sem_b)
      cs.wait()
      cd.wait()
      ws = pltpu.make_async_copy(
          as_v, os_hbm.at[pl.ds(base + j * CHUNK, CHUNK)], sem_a)
      wd = pltpu.make_async_copy(
          ad_v, od_hbm.at[pl.ds(base + j * CHUNK, CHUNK)], sem_b)
      ws.start()
      wd.start()
      ws.wait()
      wd.wait()
      return carry

    lax.fori_loop(0, ch, step, 0)

  return k(gs, gd, srci, dsti)


def _sc_scatter(rows, dsti, zeros_n, n_pad, e_pad):
  ch = e_pad // (NW * CHUNK)
  per_w = ch * CHUNK
  mesh = plsc.VectorSubcoreMesh(core_axis_name="c", subcore_axis_name="s")

  @functools.partial(
      pl.kernel,
      out_type=jax.ShapeDtypeStruct((2, n_pad, HID), F32),
      mesh=mesh,
      scratch_types=[
          pltpu.VMEM((CHUNK,), jnp.int32),
          pltpu.VMEM((CHUNK, HID), F32),
          pltpu.VMEM_SHARED((n_pad, HID), F32),
          pltpu.SemaphoreType.DMA,
          pltpu.SemaphoreType.DMA,
      ])
  def k(r_hbm, di_hbm, z_hbm, out_hbm, di_v, fb_v, acc_sh, sem_a, sem_b):
    cid = lax.axis_index("c")
    sid = lax.axis_index("s")
    wid = sid * 2 + cid

    @pl.when(sid == 0)
    def _():
      pltpu.sync_copy(z_hbm, acc_sh)

    plsc.subcore_barrier()
    base = wid * per_w

    def step(j, carry):
      ra = pltpu.async_copy(r_hbm.at[pl.ds(base + j * CHUNK, CHUNK)], fb_v,
                            sem_a)
      ri = pltpu.async_copy(di_hbm.at[wid, j], di_v, sem_b)
      ra.wait()
      ri.wait()
      pltpu.sync_copy(fb_v, acc_sh.at[di_v], add=True)
      return carry

    lax.fori_loop(0, ch, step, 0)
    plsc.subcore_barrier()

    @pl.when(sid == 0)
    def _():
      pltpu.sync_copy(acc_sh, out_hbm.at[cid])

  return k(rows, dsti, zeros_n)


def kernel(x, edge_attr, edge_index, params):
  n = x.shape[0]
  e = edge_attr.shape[0]
  n_pad = _rup(n, BLK_N)
  e_pad = _rup(e, NW * CHUNK)
  pad_node = n_pad - 1

  src = edge_index[0].astype(jnp.int32)
  dst = edge_index[1].astype(jnp.int32)
  srci = jnp.full((e_pad,), pad_node, jnp.int32).at[:e].set(src)
  dsti = jnp.full((e_pad,), pad_node, jnp.int32).at[:e].set(dst)
  srci = srci.reshape(NW, -1, CHUNK)
  dsti = dsti.reshape(NW, -1, CHUNK)
  zeros_n = jnp.zeros((n_pad, HID), F32)

  x_p = jnp.zeros((n_pad, 8), F32).at[:n, :2].set(x)
  ea_p = jnp.zeros((e_pad, 8), F32).at[:e, :3].set(edge_attr)

  def prep(p, in_pad=None):
    w0 = p['w0']
    if in_pad is not None:
      w0 = jnp.zeros((in_pad, w0.shape[1]), F32).at[:w0.shape[0]].set(w0)
    out = [w0, p['b0'].reshape(1, -1), p['w1'], p['b1'].reshape(1, -1),
           p['w2'], p['b2'].reshape(1, -1)]
    if 'ln_g' in p:
      out += [p['ln_g'].reshape(1, -1), p['ln_b'].reshape(1, -1)]
    return out

  xh, = _rows(_enc_body, [x_p], prep(params['enc_cb'], 8), [HID], BLK_N)
  eh, = _rows(_enc_body, [ea_p], prep(params['enc_eb'], 8), [HID], BLK_E)
  x0, e0 = xh, eh

  mp = len(params['blocks'])
  for i, blk in enumerate(params['blocks']):
    skip = (i == mp - mp // 2)
    cbw = prep(blk['cb'])
    ebw = prep(blk['eb'])
    w0x, w0a = cbw[0][:HID], cbw[0][HID:]
    w0s, w0d, w0e = ebw[0][:HID], ebw[0][HID:2 * HID], ebw[0][2 * HID:]

    agg = _sc_scatter(eh, dsti, zeros_n, n_pad, e_pad)
    row_ins = [xh, agg[0], agg[1]] + ([x0] if skip else [])
    xh, gs, gd = _rows(
        _make_cb_body(skip), row_ins,
        [w0x, w0a] + cbw[1:] + [w0s, w0d], [HID, HID, HID], BLK_N)

    gth_s, gth_d = _sc_gather(gs, gd, srci, dsti, e_pad)

    row_ins = [eh, gth_s, gth_d] + ([e0] if skip else [])
    last = (i == mp - 1)
    if not last:
      eh, = _rows(_make_edge_body(skip, False), row_ins, [w0e] + ebw[1:],
                  [HID], BLK_E)
    else:
      dw = prep(params['dec'])
      odim = dw[4].shape[1]
      opad = _rup(odim, 8)
      dw[4] = jnp.zeros((HID, opad), F32).at[:, :odim].set(dw[4])
      dw[5] = jnp.zeros((1, opad), F32).at[:, :odim].set(dw[5])
      out_p, = _rows(_make_edge_body(skip, True), row_ins,
                     [w0e] + ebw[1:] + dw, [opad], BLK_E)
  return out_p[:e, :odim]


# speedup vs baseline: 1.0400x; 1.0023x over previous
import functools

import jax
import jax.numpy as jnp
from jax import lax
from jax.experimental import pallas as pl
from jax.experimental.pallas import tpu as pltpu
import jax.experimental.pallas.tpu_sc as plsc

F32 = jnp.float32
HID = 128
H2 = HID // 2
NW = 32
CHUNK = 128
BLK_E = 2048
BLK_N = 1024


def _cdiv(a, b):
  return (a + b - 1) // b


def _rup(a, b):
  return _cdiv(a, b) * b


def _silu(h):
  return h * jax.nn.sigmoid(h)


def _ln(h, g, b):
  mu = jnp.mean(h, axis=-1, keepdims=True)
  d = h - mu
  var = jnp.mean(d * d, axis=-1, keepdims=True)
  return d * lax.rsqrt(var + 1e-5) * g + b


def _dot(a, b):
  return jnp.dot(a, b, preferred_element_type=F32)


def _rows(body, row_ins, w_ins, out_dims, blk):
  nrows = row_ins[0].shape[0]
  grid = (nrows // blk,)
  in_specs = (
      [pl.BlockSpec((blk, a.shape[1]), lambda i: (i, 0)) for a in row_ins]
      + [pl.BlockSpec(w.shape, lambda i, nd=w.ndim: (0,) * nd) for w in w_ins]
  )
  out_specs = [pl.BlockSpec((blk, d), lambda i: (i, 0)) for d in out_dims]
  out_shape = [jax.ShapeDtypeStruct((nrows, d), F32) for d in out_dims]
  outs = pl.pallas_call(
      body, grid=grid, in_specs=in_specs, out_specs=out_specs,
      out_shape=out_shape)(*row_ins, *w_ins)
  return outs


def _enc_body(x_ref, w0, b0, w1, b1, w2, b2, g, b, out_ref):
  h = _silu(_dot(x_ref[...], w0[...]) + b0[...])
  h = _silu(_dot(h, w1[...]) + b1[...])
  h = _dot(h, w2[...]) + b2[...]
  out_ref[...] = _ln(h, g[...], b[...])


def _make_cb_body(skip):
  def body(*refs):
    xh_ref, a0_ref, a1_ref = refs[:3]
    rest = refs[3:]
    if skip:
      x0_ref, rest = rest[0], rest[1:]
    w0x, w0a, b0, w1, b1, w2, b2, g, b, w0s, w0d = rest[:11]
    xh_out, gs_out, gd_out = rest[11:]
    xh = xh_ref[...]
    agg = a0_ref[...] + a1_ref[...]
    aggf = 0.5 * (agg[:, :H2] + agg[:, H2:])
    h = _silu(_dot(xh, w0x[...]) + _dot(aggf, w0a[...]) + b0[...])
    h = _silu(_dot(h, w1[...]) + b1[...])
    h = _dot(h, w2[...]) + b2[...]
    xn = _ln(h, g[...], b[...])
    out = xh + xn
    if skip:
      out = out + x0_ref[...]
    xh_out[...] = out
    gs_out[...] = _dot(xn, w0s[...])
    gd_out[...] = _dot(xn, w0d[...])
  return body


def _make_edge_body(skip, decode):
  def body(*refs):
    eh_ref, gs_ref, gd_ref = refs[:3]
    rest = refs[3:]
    if skip:
      e0_ref, rest = rest[0], rest[1:]
    w0e, b0, w1, b1, w2, b2, g, b = rest[:8]
    rest = rest[8:]
    eh = eh_ref[...]
    h = _silu(gs_ref[...] + gd_ref[...] + _dot(eh, w0e[...]) + b0[...])
    h = _silu(_dot(h, w1[...]) + b1[...])
    h = _dot(h, w2[...]) + b2[...]
    en = _ln(h, g[...], b[...])
    out = eh + en
    if skip:
      out = out + e0_ref[...]
    if decode:
      dw0, db0, dw1, db1, dw2, db2, out_ref = rest
      h = _silu(_dot(out, dw0[...]) + db0[...])
      h = _silu(_dot(h, dw1[...]) + db1[...])
      out_ref[...] = _dot(h, dw2[...]) + db2[...]
    else:
      rest[0][...] = out
  return body


def _sc_gather(gs, gd, srci, dsti, e_pad):
  ch = e_pad // (NW * CHUNK)
  per_w = ch * CHUNK
  mesh = plsc.VectorSubcoreMesh(core_axis_name="c", subcore_axis_name="s")

  @functools.partial(
      pl.kernel,
      out_type=[jax.ShapeDtypeStruct((e_pad, HID), F32)] * 2,
      mesh=mesh,
      scratch_types=[
          pltpu.VMEM((ch, CHUNK), jnp.int32),
          pltpu.VMEM((ch, CHUNK), jnp.int32),
          pltpu.VMEM((CHUNK, HID), F32),
          pltpu.VMEM((CHUNK, HID), F32),
          pltpu.SemaphoreType.DMA,
          pltpu.SemaphoreType.DMA,
      ])
  def k(gs_hbm, gd_hbm, si_hbm, di_hbm, os_hbm, od_hbm,
        si_v, di_v, as_v, ad_v, sem_a, sem_b):
    cid = lax.axis_index("c")
    sid = lax.axis_index("s")
    wid = sid * 2 + cid
    pltpu.sync_copy(si_hbm.at[wid], si_v)
    pltpu.sync_copy(di_hbm.at[wid], di_v)
    base = wid * per_w

    def step(j, carry):
      cs = pltpu.async_copy(gs_hbm.at[si_v.at[j]], as_v, sem_a)
      cd = pltpu.async_copy(gd_hbm.at[di_v.at[j]], ad_v, sem_b)
      cs.wait()
      cd.wait()
      ws = pltpu.make_async_copy(
          as_v, os_hbm.at[pl.ds(base + j * CHUNK, CHUNK)], sem_a)
      wd = pltpu.make_async_copy(
          ad_v, od_hbm.at[pl.ds(base + j * CHUNK, CHUNK)], sem_b)
      ws.start()
      wd.start()
      ws.wait()
      wd.wait()
      return carry

    lax.fori_loop(0, ch, step, 0)

  return k(gs, gd, srci, dsti)


def _sc_scatter(rows, dsti, zeros_n, n_pad, e_pad):
  ch = e_pad // (NW * CHUNK)
  per_w = ch * CHUNK
  mesh = plsc.VectorSubcoreMesh(core_axis_name="c", subcore_axis_name="s")

  @functools.partial(
      pl.kernel,
      out_type=jax.ShapeDtypeStruct((2, n_pad, HID), F32),
      mesh=mesh,
      scratch_types=[
          pltpu.VMEM((CHUNK,), jnp.int32),
          pltpu.VMEM((CHUNK, HID), F32),
          pltpu.VMEM_SHARED((n_pad, HID), F32),
          pltpu.SemaphoreType.DMA,
          pltpu.SemaphoreType.DMA,
      ])
  def k(r_hbm, di_hbm, z_hbm, out_hbm, di_v, fb_v, acc_sh, sem_a, sem_b):
    cid = lax.axis_index("c")
    sid = lax.axis_index("s")
    wid = sid * 2 + cid

    @pl.when(sid == 0)
    def _():
      pltpu.sync_copy(z_hbm, acc_sh)

    plsc.subcore_barrier()
    base = wid * per_w

    def step(j, carry):
      ra = pltpu.async_copy(r_hbm.at[pl.ds(base + j * CHUNK, CHUNK)], fb_v,
                            sem_a)
      ri = pltpu.async_copy(di_hbm.at[wid, j], di_v, sem_b)
      ra.wait()
      ri.wait()
      pltpu.sync_copy(fb_v, acc_sh.at[di_v], add=True)
      return carry

    lax.fori_loop(0, ch, step, 0)
    plsc.subcore_barrier()

    @pl.when(sid == 0)
    def _():
      pltpu.sync_copy(acc_sh, out_hbm.at[cid])

  return k(rows, dsti, zeros_n)


def kernel(x, edge_attr, edge_index, params):
  n = x.shape[0]
  e = edge_attr.shape[0]
  n_pad = _rup(n, BLK_N)
  e_pad = _rup(e, NW * CHUNK)
  pad_node = n_pad - 1

  src = edge_index[0].astype(jnp.int32)
  dst = edge_index[1].astype(jnp.int32)
  srci = jnp.full((e_pad,), pad_node, jnp.int32).at[:e].set(src)
  dsti = jnp.full((e_pad,), pad_node, jnp.int32).at[:e].set(dst)
  srci = srci.reshape(NW, -1, CHUNK)
  dsti = dsti.reshape(NW, -1, CHUNK)
  zeros_n = jnp.zeros((n_pad, HID), F32)

  x_p = jnp.zeros((n_pad, 8), F32).at[:n, :2].set(x)
  ea_p = jnp.zeros((e_pad, 8), F32).at[:e, :3].set(edge_attr)

  def prep(p, in_pad=None):
    w0 = p['w0']
    if in_pad is not None:
      w0 = jnp.zeros((in_pad, w0.shape[1]), F32).at[:w0.shape[0]].set(w0)
    out = [w0, p['b0'].reshape(1, -1), p['w1'], p['b1'].reshape(1, -1),
           p['w2'], p['b2'].reshape(1, -1)]
    if 'ln_g' in p:
      out += [p['ln_g'].reshape(1, -1), p['ln_b'].reshape(1, -1)]
    return out

  xh, = _rows(_enc_body, [x_p], prep(params['enc_cb'], 8), [HID], BLK_N)
  eh, = _rows(_enc_body, [ea_p], prep(params['enc_eb'], 8), [HID], BLK_E)
  x0, e0 = xh, eh

  mp = len(params['blocks'])
  for i, blk in enumerate(params['blocks']):
    skip = (i == mp - mp // 2)
    cbw = prep(blk['cb'])
    ebw = prep(blk['eb'])
    w0x, w0a = cbw[0][:HID], cbw[0][HID:]
    w0s, w0d, w0e = ebw[0][:HID], ebw[0][HID:2 * HID], ebw[0][2 * HID:]

    agg = _sc_scatter(eh, dsti, zeros_n, n_pad, e_pad)
    row_ins = [xh, agg[0], agg[1]] + ([x0] if skip else [])
    xh, gs, gd = _rows(
        _make_cb_body(skip), row_ins,
        [w0x, w0a] + cbw[1:] + [w0s, w0d], [HID, HID, HID], BLK_N)

    gth_s, gth_d = _sc_gather(gs, gd, srci, dsti, e_pad)

    row_ins = [eh, gth_s, gth_d] + ([e0] if skip else [])
    last = (i == mp - 1)
    if not last:
      eh, = _rows(_make_edge_body(skip, False), row_ins, [w0e] + ebw[1:],
                  [HID], BLK_E)
    else:
      dw = prep(params['dec'])
      odim = dw[4].shape[1]
      opad = _rup(odim, 8)
      dw[4] = jnp.zeros((HID, opad), F32).at[:, :odim].set(dw[4])
      dw[5] = jnp.zeros((1, opad), F32).at[:, :odim].set(dw[5])
      out_p, = _rows(_make_edge_body(skip, True), row_ins,
                     [w0e] + ebw[1:] + dw, [opad], BLK_E)
  return out_p[:e, :odim]
